# baseline (speedup 1.0000x reference)
import functools

import jax
import jax.numpy as jnp
from jax import lax
from jax.experimental import pallas as pl
from jax.experimental.pallas import tpu as pltpu

K = 512
D = 64


def _exact_dist(x, c):
    t = x - c
    sq = t * t
    s = None
    for r in range(8):
        lo = 8 * r
        a = sq[:, lo:lo + 4] + sq[:, lo + 4:lo + 8]
        b = a[:, 0:2] + a[:, 2:4]
        blk = b[:, 0:1] + b[:, 1:2]
        s = blk if s is None else s + blk
    return s


def _vq_tc_body(x_ref, cbt_ref, cb_ref, out_ref):
    x = x_ref[...]
    cbt = cbt_ref[...]
    cb = cb_ref[...]
    xc = lax.dot_general(
        x, cbt, (((1,), (0,)), ((), ())),
        preferred_element_type=jnp.float32,
        precision=lax.Precision.HIGHEST,
    )
    cnorm = jnp.sum(cbt * cbt, axis=0)[None, :]
    scores = cnorm - 2.0 * xc
    iota = lax.broadcasted_iota(jnp.int32, scores.shape, 1)
    m1 = jnp.min(scores, axis=1, keepdims=True)
    tk1 = jnp.min(jnp.where(scores == m1, iota, K), axis=1, keepdims=True)
    masked = jnp.where(iota == tk1, jnp.inf, scores)
    m2 = jnp.min(masked, axis=1, keepdims=True)
    tk2 = jnp.min(jnp.where(masked == m2, iota, K), axis=1, keepdims=True)
    oh1 = (iota == tk1).astype(jnp.float32)
    oh2 = (iota == tk2).astype(jnp.float32)
    c1 = lax.dot_general(
        oh1, cb, (((1,), (0,)), ((), ())),
        preferred_element_type=jnp.float32, precision=lax.Precision.HIGHEST)
    c2 = lax.dot_general(
        oh2, cb, (((1,), (0,)), ((), ())),
        preferred_element_type=jnp.float32, precision=lax.Precision.HIGHEST)
    d1 = _exact_dist(x, c1)
    d2 = _exact_dist(x, c2)
    pick1 = (d1 < d2) | ((d1 == d2) & (tk1 < tk2))
    emb = jnp.where(pick1, c1, c2)
    out_ref[...] = x + (emb - x)


@functools.partial(jax.jit, static_argnames=("interpret",))
def _vq_tc(x2d, cbt, cb, interpret=False):
    n = x2d.shape[0]
    return pl.pallas_call(
        _vq_tc_body,
        out_shape=jax.ShapeDtypeStruct((n, D), jnp.float32),
        interpret=interpret,
    )(x2d, cbt, cb)


def kernel(inputs, codebook, training):
    x2d = inputs.reshape(-1, D)
    out = _vq_tc(x2d, codebook.T, codebook)
    return out.reshape(inputs.shape)
